# baseline (speedup 1.0000x reference)
import functools

import jax
import jax.numpy as jnp
from jax import lax
from jax.experimental import pallas as pl
from jax.experimental.pallas import tpu as pltpu
from jax.experimental.pallas import tpu_sc as plsc

DIM = 1024
MAX_T = 1000
BATCH = 16384

NC = 2
NS = 16
NW = NC * NS
BPW = BATCH // NW
CHUNK = 32
NCHUNK = BPW // CHUNK
NBUF = 3


def _gather_body(table_hbm, idx_hbm, out_hbm, idx_v, *bufs_and_sems):
    bufs = bufs_and_sems[:NBUF]
    gsems = bufs_and_sems[NBUF:2 * NBUF]
    osems = bufs_and_sems[2 * NBUF:3 * NBUF]

    cid = lax.axis_index("c")
    sid = lax.axis_index("s")
    wid = sid * NC + cid
    base = wid * BPW

    pltpu.sync_copy(idx_hbm.at[wid], idx_v)

    g_handles = [None] * NCHUNK
    o_handles = [None] * NBUF

    for j in range(NCHUNK + 1):
        b = j % NBUF
        if j < NCHUNK:
            if j >= NBUF:
                o_handles[b].wait()
            g_handles[j] = pltpu.async_copy(
                table_hbm.at[idx_v.at[j]], bufs[b], gsems[b])
        if j >= 1:
            pb = (j - 1) % NBUF
            g_handles[j - 1].wait()
            o_handles[pb] = pltpu.async_copy(
                bufs[pb], out_hbm.at[pl.ds(base + (j - 1) * CHUNK, CHUNK)],
                osems[pb])
    for h in o_handles:
        h.wait()


@jax.jit
def _pe_lookup(table, idx3):
    mesh = plsc.VectorSubcoreMesh(core_axis_name="c", subcore_axis_name="s")
    k = functools.partial(
        pl.kernel,
        mesh=mesh,
        out_type=jax.ShapeDtypeStruct((BATCH, DIM), jnp.float32),
        scratch_types=(
            [pltpu.VMEM((NCHUNK, CHUNK), jnp.int32)]
            + [pltpu.VMEM((CHUNK, DIM), jnp.float32) for _ in range(NBUF)]
            + [pltpu.SemaphoreType.DMA for _ in range(2 * NBUF)]
        ),
    )(_gather_body)
    return k(table, idx3)


def kernel(timestep, pe_matrix):
    table = pe_matrix.reshape(MAX_T, DIM)
    idx3 = timestep.astype(jnp.int32).reshape(NW, NCHUNK, CHUNK)
    out = _pe_lookup(table, idx3)
    return out.reshape(BATCH, 1, DIM)


# speedup vs baseline: 1.7886x; 1.7886x over previous
import functools

import jax
import jax.numpy as jnp
from jax import lax
from jax.experimental import pallas as pl
from jax.experimental.pallas import tpu as pltpu
from jax.experimental.pallas import tpu_sc as plsc

DIM = 1024
MAX_T = 1000
BATCH = 16384

NC = 2
NS = 16
NW = NC * NS
BPW = BATCH // NW
CHUNK = 32
NCHUNK = BPW // CHUNK
NBUF = 3


def _gather_body(table_hbm, idx_hbm, out_hbm, idx_v, *bufs_and_sems):
    bufs = bufs_and_sems[:NBUF]
    gsems = bufs_and_sems[NBUF:2 * NBUF]
    osems = bufs_and_sems[2 * NBUF:3 * NBUF]

    cid = lax.axis_index("c")
    sid = lax.axis_index("s")
    wid = sid * NC + cid
    base = wid * BPW

    pltpu.sync_copy(idx_hbm.at[wid], idx_v)

    g_handles = [None] * NCHUNK
    o_handles = [None] * NBUF

    for j in range(NCHUNK + 1):
        b = j % NBUF
        if j < NCHUNK:
            if j >= NBUF:
                o_handles[b].wait()
            g_handles[j] = pltpu.async_copy(
                table_hbm.at[idx_v.at[j]], bufs[b], gsems[b])
        if j >= 1:
            pb = (j - 1) % NBUF
            g_handles[j - 1].wait()
            o_handles[pb] = pltpu.async_copy(
                bufs[pb], out_hbm.at[pl.ds(base + (j - 1) * CHUNK, CHUNK)],
                osems[pb])
    for h in o_handles:
        h.wait()


@jax.jit
def _pe_lookup(table, idx3):
    mesh = plsc.VectorSubcoreMesh(core_axis_name="c", subcore_axis_name="s")
    k = functools.partial(
        pl.kernel,
        mesh=mesh,
        out_type=jax.ShapeDtypeStruct((BATCH, 1, DIM), jnp.float32),
        scratch_types=(
            [pltpu.VMEM((NCHUNK, CHUNK), jnp.int32)]
            + [pltpu.VMEM((CHUNK, 1, DIM), jnp.float32) for _ in range(NBUF)]
            + [pltpu.SemaphoreType.DMA for _ in range(2 * NBUF)]
        ),
    )(_gather_body)
    return k(table, idx3)


def kernel(timestep, pe_matrix):
    idx3 = timestep.astype(jnp.int32).reshape(NW, NCHUNK, CHUNK)
    return _pe_lookup(pe_matrix, idx3)


# speedup vs baseline: 1.7940x; 1.0030x over previous
import functools

import jax
import jax.numpy as jnp
from jax import lax
from jax.experimental import pallas as pl
from jax.experimental.pallas import tpu as pltpu
from jax.experimental.pallas import tpu_sc as plsc

DIM = 1024
MAX_T = 1000
BATCH = 16384

NC = 2
NS = 16
NW = NC * NS
BPW = BATCH // NW
CHUNK = 32
NCHUNK = BPW // CHUNK
NBUF = 3


def _gather_body(table_hbm, idx_hbm, out_hbm, idx_v, *bufs_and_sems):
    bufs = bufs_and_sems[:NBUF]
    gsems = bufs_and_sems[NBUF:2 * NBUF]
    osems = bufs_and_sems[2 * NBUF:3 * NBUF]

    cid = lax.axis_index("c")
    sid = lax.axis_index("s")
    wid = sid * NC + cid
    base = wid * BPW

    pltpu.sync_copy(idx_hbm.at[pl.ds(base, BPW)], idx_v)

    g_handles = [None] * NCHUNK
    o_handles = [None] * NBUF

    for j in range(NCHUNK + 1):
        b = j % NBUF
        if j < NCHUNK:
            if j >= NBUF:
                o_handles[b].wait()
            g_handles[j] = pltpu.async_copy(
                table_hbm.at[idx_v.at[pl.ds(j * CHUNK, CHUNK)]],
                bufs[b], gsems[b])
        if j >= 1:
            pb = (j - 1) % NBUF
            g_handles[j - 1].wait()
            o_handles[pb] = pltpu.async_copy(
                bufs[pb], out_hbm.at[pl.ds(base + (j - 1) * CHUNK, CHUNK)],
                osems[pb])
    for h in o_handles:
        h.wait()


@jax.jit
def _pe_lookup(table, idx3):
    mesh = plsc.VectorSubcoreMesh(core_axis_name="c", subcore_axis_name="s")
    k = functools.partial(
        pl.kernel,
        mesh=mesh,
        out_type=jax.ShapeDtypeStruct((BATCH, 1, DIM), jnp.float32),
        scratch_types=(
            [pltpu.VMEM((BPW,), jnp.int32)]
            + [pltpu.VMEM((CHUNK, 1, DIM), jnp.float32) for _ in range(NBUF)]
            + [pltpu.SemaphoreType.DMA for _ in range(2 * NBUF)]
        ),
    )(_gather_body)
    return k(table, idx3)


def kernel(timestep, pe_matrix):
    return _pe_lookup(pe_matrix, timestep.astype(jnp.int32))


# speedup vs baseline: 1.8205x; 1.0148x over previous
import functools

import jax
import jax.numpy as jnp
from jax import lax
from jax.experimental import pallas as pl
from jax.experimental.pallas import tpu as pltpu
from jax.experimental.pallas import tpu_sc as plsc

DIM = 1024
MAX_T = 1000
BATCH = 16384

NC = 2
NS = 16
NW = NC * NS
BPW = BATCH // NW
CHUNK = 48
CHUNK_SIZES = [48] * 10 + [32]
NCHUNK = len(CHUNK_SIZES)
CHUNK_OFFS = [sum(CHUNK_SIZES[:i]) for i in range(NCHUNK)]
NBUF = 2


def _gather_body(table_hbm, idx_hbm, out_hbm, idx_v, *bufs_and_sems):
    bufs = bufs_and_sems[:NBUF]
    gsems = bufs_and_sems[NBUF:2 * NBUF]
    osems = bufs_and_sems[2 * NBUF:3 * NBUF]

    cid = lax.axis_index("c")
    sid = lax.axis_index("s")
    wid = sid * NC + cid
    base = wid * BPW

    pltpu.sync_copy(idx_hbm.at[pl.ds(base, BPW)], idx_v)

    g_handles = [None] * NCHUNK
    o_handles = [None] * NBUF

    for j in range(NCHUNK + 1):
        b = j % NBUF
        if j < NCHUNK:
            if j >= NBUF:
                o_handles[b].wait()
            n = CHUNK_SIZES[j]
            dst = bufs[b] if n == CHUNK else bufs[b].at[pl.ds(0, n)]
            g_handles[j] = pltpu.async_copy(
                table_hbm.at[idx_v.at[pl.ds(CHUNK_OFFS[j], n)]],
                dst, gsems[b])
        if j >= 1:
            pb = (j - 1) % NBUF
            g_handles[j - 1].wait()
            pn = CHUNK_SIZES[j - 1]
            src = bufs[pb] if pn == CHUNK else bufs[pb].at[pl.ds(0, pn)]
            o_handles[pb] = pltpu.async_copy(
                src, out_hbm.at[pl.ds(base + CHUNK_OFFS[j - 1], pn)],
                osems[pb])
    for h in o_handles:
        h.wait()


@jax.jit
def _pe_lookup(table, idx3):
    mesh = plsc.VectorSubcoreMesh(core_axis_name="c", subcore_axis_name="s")
    k = functools.partial(
        pl.kernel,
        mesh=mesh,
        out_type=jax.ShapeDtypeStruct((BATCH, 1, DIM), jnp.float32),
        scratch_types=(
            [pltpu.VMEM((BPW,), jnp.int32)]
            + [pltpu.VMEM((CHUNK, 1, DIM), jnp.float32) for _ in range(NBUF)]
            + [pltpu.SemaphoreType.DMA for _ in range(2 * NBUF)]
        ),
    )(_gather_body)
    return k(table, idx3)


def kernel(timestep, pe_matrix):
    return _pe_lookup(pe_matrix, timestep.astype(jnp.int32))
